# Initial kernel scaffold; baseline (speedup 1.0000x reference)
import jax
import jax.numpy as jnp
from jax.experimental import pallas as pl


def kernel(node_feat, W0, b0, W1, b1, W2, b2, W3, b3, W4, b4, W5, b5, W6, b6, W7, b7, conv1_w, conv1_b, conv2_w, conv2_b, out_w, out_b, inc_node, inc_hedge):
    raise NotImplementedError("write your pallas kernel here")


# speedup vs baseline: 2.0624x; 2.0624x over previous
import functools

import jax
import jax.numpy as jnp
from jax import lax
from jax.experimental import pallas as pl
from jax.experimental.pallas import tpu as pltpu
from jax.experimental.pallas import tpu_sc as plsc

_N = 10000
_E = 320000
_G = 100
_NPG = 100
_K = 30

_NC, _NS = 2, 16
_NW = _NC * _NS
_RNG = 312
_ACCR = 336
_DUM = 328
_MAXE = 12288
_NCHK = _MAXE // 128
_CHE = 2000
_NSC = _E // _CHE

_f32 = jnp.float32
_i32 = jnp.int32

_SC_PARAMS = pltpu.CompilerParams(use_tc_tiling_on_sc=False,
                                  needs_layout_passes=False)


def _mesh():
    return plsc.VectorSubcoreMesh(core_axis_name="c", subcore_axis_name="s",
                                  num_cores=_NC, num_subcores=_NS)


def _worker_bounds():
    c = lax.axis_index("c")
    s = lax.axis_index("s")
    w = c * _NS + s
    lo = w * _RNG
    hi = jnp.where(w == _NW - 1, _N, lo + _RNG)
    return w, lo, hi


def _compact_body(srch, dsth, csF, cdF, cntF, csN, cdN, cntN, hdeg, ndeg,
                  sbuf, dbuf, csF_v, cdF_v, csN_v, cdN_v, degF, degN, cnt_v):
    w, lo, hi = _worker_bounds()
    ii = lax.iota(_i32, 16)
    z16 = jnp.zeros((16,), _f32)
    ones16 = z16 + 1.0

    def zero_deg(i, _):
        degF[pl.ds(i * 16, 16)] = z16
        degN[pl.ds(i * 16, 16)] = z16
        return 0

    lax.fori_loop(0, _ACCR // 16, zero_deg, 0)

    def scan_chunk(kc, ptrs):
        pltpu.sync_copy(srch.at[pl.ds(kc * _CHE, _CHE)], sbuf)
        pltpu.sync_copy(dsth.at[pl.ds(kc * _CHE, _CHE)], dbuf)

        def scan_vreg(j, ptrs):
            pF, pN = ptrs
            n16 = sbuf[pl.ds(j * 16, 16)]
            h16 = dbuf[pl.ds(j * 16, 16)]
            mF = (h16 >= lo) & (h16 < hi)
            hloc = h16 - lo
            pFu = jnp.minimum(pF, _MAXE - 16)
            plsc.store_compressed(csF_v.at[pl.ds(pFu, 16)], n16, mask=mF)
            plsc.store_compressed(cdF_v.at[pl.ds(pFu, 16)], hloc, mask=mF)
            plsc.addupdate_scatter(degF, [hloc], ones16, mask=mF)
            pF = pF + lax.reduce_max(plsc.all_reduce_population_count(mF),
                                     axes=(0,))
            mN = (n16 >= lo) & (n16 < hi)
            nloc = n16 - lo
            pNu = jnp.minimum(pN, _MAXE - 16)
            plsc.store_compressed(csN_v.at[pl.ds(pNu, 16)], h16, mask=mN)
            plsc.store_compressed(cdN_v.at[pl.ds(pNu, 16)], nloc, mask=mN)
            plsc.addupdate_scatter(degN, [nloc], ones16, mask=mN)
            pN = pN + lax.reduce_max(plsc.all_reduce_population_count(mN),
                                     axes=(0,))
            return (pF, pN)

        return lax.fori_loop(0, _CHE // 16, scan_vreg, ptrs)

    pF, pN = lax.fori_loop(0, _NSC, scan_chunk, (jnp.int32(0), jnp.int32(0)))

    zi16 = ii * 0
    dum16 = zi16 + _DUM
    for (p, cs_v, cd_v, cnt_h) in ((pF, csF_v, cdF_v, cntF),
                                   (pN, csN_v, cdN_v, cntN)):
        pe = jnp.minimum(p, _MAXE - 128)
        for t in range(8):
            cs_v[pl.ds(pe + t * 16, 16)] = zi16
            cd_v[pl.ds(pe + t * 16, 16)] = dum16
        nch = (pe + 128) // 128
        cnt_v[...] = jnp.broadcast_to(nch, (16,))
        pltpu.sync_copy(cnt_v, cnt_h.at[w])

    pltpu.sync_copy(csF_v, csF.at[w])
    pltpu.sync_copy(cdF_v, cdF.at[w])
    pltpu.sync_copy(csN_v, csN.at[w])
    pltpu.sync_copy(cdN_v, cdN.at[w])
    pltpu.sync_copy(degF.at[pl.ds(0, _RNG)], hdeg.at[pl.ds(lo, _RNG)])
    pltpu.sync_copy(degN.at[pl.ds(0, _RNG)], ndeg.at[pl.ds(lo, _RNG)])

    @pl.when(w == _NW - 1)
    def _():
        t0 = _RNG * _NW
        pltpu.sync_copy(degF.at[pl.ds(_RNG, 16)], hdeg.at[pl.ds(t0, 16)])
        pltpu.sync_copy(degN.at[pl.ds(_RNG, 16)], ndeg.at[pl.ds(t0, 16)])


@functools.lru_cache(maxsize=None)
def _make_compact():
    out_type = [
        jax.ShapeDtypeStruct((_NW, _MAXE), _i32),
        jax.ShapeDtypeStruct((_NW, _MAXE), _i32),
        jax.ShapeDtypeStruct((_NW, 16), _i32),
        jax.ShapeDtypeStruct((_NW, _MAXE), _i32),
        jax.ShapeDtypeStruct((_NW, _MAXE), _i32),
        jax.ShapeDtypeStruct((_NW, 16), _i32),
        jax.ShapeDtypeStruct((_N,), _f32),
        jax.ShapeDtypeStruct((_N,), _f32),
    ]
    scratch = [
        pltpu.VMEM((_CHE,), _i32),
        pltpu.VMEM((_CHE,), _i32),
        pltpu.VMEM((_MAXE,), _i32),
        pltpu.VMEM((_MAXE,), _i32),
        pltpu.VMEM((_MAXE,), _i32),
        pltpu.VMEM((_MAXE,), _i32),
        pltpu.VMEM((_ACCR,), _f32),
        pltpu.VMEM((_ACCR,), _f32),
        pltpu.VMEM((16,), _i32),
    ]
    return functools.partial(pl.kernel, out_type=out_type, mesh=_mesh(),
                             scratch_types=scratch,
                             compiler_params=_SC_PARAMS)(_compact_body)


def _seg_body(width, table, csrc3, cdst, cnth, out,
              csrc_v, cdst_v, cnt_v, rows_v, acc, sem):
    w, lo, hi = _worker_bounds()
    ii = lax.iota(_i32, 16)
    z16 = jnp.zeros((16,), _f32)

    pltpu.sync_copy(csrc3.at[w], csrc_v)
    pltpu.sync_copy(cdst.at[w], cdst_v)
    pltpu.sync_copy(cnth.at[w], cnt_v)
    nch = lax.reduce_max(cnt_v[...], axes=(0,))

    jb = width // 16 if width >= 16 else 1

    def zero_acc(r, _):
        for t in range(jb):
            if width >= 16:
                acc[r, pl.ds(t * 16, 16)] = z16
            else:
                acc[pl.ds(r * 16, 16)] = z16
        return 0

    lax.fori_loop(0, _ACCR if width >= 16 else _ACCR // 16, zero_acc, 0)

    def chunk(kc, _):
        pltpu.async_copy(table.at[csrc_v.at[kc]], rows_v, sem).wait()
        for v in range(8):
            dloc = cdst_v[pl.ds(kc * 128 + v * 16, 16)]
            rbase = ii + v * 16
            if width == 32:
                for j in range(width):
                    j16 = ii * 0 + j
                    vals = plsc.load_gather(rows_v, [rbase, j16])
                    plsc.addupdate_scatter(acc, [dloc, j16], vals)
            else:

                def cols(jg, _):
                    for t in range(16):
                        j16 = ii * 0 + (jg * 16 + t)
                        vals = plsc.load_gather(rows_v, [rbase, j16])
                        plsc.addupdate_scatter(acc, [dloc, j16], vals)
                    return 0

                lax.fori_loop(0, width // 16, cols, 0)
        return 0

    lax.fori_loop(0, nch, chunk, 0)

    pltpu.sync_copy(acc.at[pl.ds(0, _RNG)], out.at[pl.ds(lo, _RNG)])

    @pl.when(w == _NW - 1)
    def _():
        t0 = _RNG * _NW
        pltpu.sync_copy(acc.at[pl.ds(_RNG, 16)], out.at[pl.ds(t0, 16)])


def _seg1_body(table, csrcf, cdst, cnth, out,
               tbl_v, csrc_v, cdst_v, cnt_v, acc):
    w, lo, hi = _worker_bounds()
    ii = lax.iota(_i32, 16)
    z16 = jnp.zeros((16,), _f32)

    pltpu.sync_copy(table, tbl_v)
    pltpu.sync_copy(csrcf.at[w], csrc_v)
    pltpu.sync_copy(cdst.at[w], cdst_v)
    pltpu.sync_copy(cnth.at[w], cnt_v)
    nch = lax.reduce_max(cnt_v[...], axes=(0,))

    def zero_acc(r, _):
        acc[pl.ds(r * 16, 16)] = z16
        return 0

    lax.fori_loop(0, _ACCR // 16, zero_acc, 0)

    def chunk(kc, _):
        for v in range(8):
            s16 = csrc_v[pl.ds(kc * 128 + v * 16, 16)]
            dloc = cdst_v[pl.ds(kc * 128 + v * 16, 16)]
            vals = plsc.load_gather(tbl_v, [s16])
            plsc.addupdate_scatter(acc, [dloc], vals)
        return 0

    lax.fori_loop(0, nch, chunk, 0)

    pltpu.sync_copy(acc.at[pl.ds(0, _RNG)], out.at[pl.ds(lo, _RNG)])

    @pl.when(w == _NW - 1)
    def _():
        t0 = _RNG * _NW
        pltpu.sync_copy(acc.at[pl.ds(_RNG, 16)], out.at[pl.ds(t0, 16)])


@functools.lru_cache(maxsize=None)
def _make_seg(width):
    if width == 1:
        scratch = [
            pltpu.VMEM((_N,), _f32),
            pltpu.VMEM((_MAXE,), _i32),
            pltpu.VMEM((_MAXE,), _i32),
            pltpu.VMEM((16,), _i32),
            pltpu.VMEM((_ACCR,), _f32),
        ]
        return functools.partial(
            pl.kernel, out_type=jax.ShapeDtypeStruct((_N,), _f32),
            mesh=_mesh(), scratch_types=scratch,
            compiler_params=_SC_PARAMS)(_seg1_body)
    scratch = [
        pltpu.VMEM((_NCHK, 128), _i32),
        pltpu.VMEM((_MAXE,), _i32),
        pltpu.VMEM((16,), _i32),
        pltpu.VMEM((128, width), _f32),
        pltpu.VMEM((_ACCR, width), _f32),
        pltpu.SemaphoreType.DMA,
    ]
    return functools.partial(
        pl.kernel, out_type=jax.ShapeDtypeStruct((_N, width), _f32),
        mesh=_mesh(), scratch_types=scratch,
        compiler_params=_SC_PARAMS)(functools.partial(_seg_body, width))


def _tc(fn, out_shape, *args, scratch_shapes=()):
    return pl.pallas_call(fn, out_shape=out_shape,
                          scratch_shapes=list(scratch_shapes))(*args)


def _layer_body(pool, wt, b, deg, h_out):
    sizes = deg[...][:, None] + 1.0
    p = jnp.dot(pool[...], wt[...], preferred_element_type=_f32)
    h_out[...] = jnp.tanh((p + b[...][None, :]) / sizes)


def _head_a_body(pool7, w7t, b7, ndeg, h2, h4, h6, c1a, c1b_, c1c, c1d, cb,
                 h8_out, y_out):
    ns = ndeg[...][:, None] + 1.0
    p = pool7[...] * w7t[...]
    h8 = jnp.tanh((p + b7[...][None, :]) / ns)
    h8_out[...] = h8
    y = (jnp.dot(h2[...], c1a[...], preferred_element_type=_f32)
         + jnp.dot(h4[...], c1b_[...], preferred_element_type=_f32)
         + jnp.dot(h6[...], c1c[...], preferred_element_type=_f32)
         + h8 * c1d[...]
         + cb[...][None, :])
    y_out[...] = jnp.maximum(y, 0.0)


def _head_b_body(scm, y3, w2r, b2, wd, ob, out, p1s):
    vals = scm[...]
    y = y3[...]
    ii = lax.broadcasted_iota(_i32, (_G, _NPG), 1)
    tmp = None
    for k in range(_K):
        mx = jnp.max(vals, axis=1, keepdims=True)
        sel = jnp.min(jnp.where(vals == mx, ii, _NPG), axis=1, keepdims=True)
        m = (ii == sel)
        ysel = jnp.sum(m.astype(_f32)[:, :, None] * y, axis=1)
        vals = jnp.where(m, -2.0, vals)
        if k % 2 == 0:
            tmp = ysel
        else:
            j = k // 2
            p1s[(j * _G):(j + 1) * _G, :] = jnp.maximum(tmp, ysel)
    c2 = jnp.zeros((11 * _G, 32), _f32)
    for t in range(5):
        st = p1s[(t * _G):(t * _G + 11 * _G), :]
        c2 = c2 + jnp.dot(st, w2r[(t * 16):(t + 1) * 16, :],
                          preferred_element_type=_f32)
    c2 = jnp.maximum(c2 + b2[...][None, :], 0.0)
    acc = jnp.zeros((_G, 2), _f32)
    for mm in range(11):
        acc = acc + jnp.dot(c2[(mm * _G):(mm + 1) * _G, :],
                            wd[(mm * 32):(mm + 1) * 32, :],
                            preferred_element_type=_f32)
    out[...] = jnp.maximum(acc + ob[...][None, :], 0.0)


def kernel(node_feat, W0, b0, W1, b1, W2, b2, W3, b3, W4, b4, W5, b5,
           W6, b6, W7, b7, conv1_w, conv1_b, conv2_w, conv2_b,
           out_w, out_b, inc_node, inc_hedge):
    src = inc_node.astype(_i32)
    dst = inc_hedge.astype(_i32)

    (csF, cdF, cntF, csN, cdN, cntN, hdeg, ndeg) = _make_compact()(src, dst)
    csF3 = csF.reshape(_NW, _NCHK, 128)
    csN3 = csN.reshape(_NW, _NCHK, 128)

    seg128 = _make_seg(128)
    seg32 = _make_seg(32)
    seg1 = _make_seg(1)

    hshape = jax.ShapeDtypeStruct((_N, 32), _f32)

    pool0 = seg128(node_feat, csF3, cdF, cntF)
    h1 = _tc(_layer_body, hshape, pool0, W0.T, b0, hdeg)
    pool1 = seg32(h1, csN3, cdN, cntN)
    h2 = _tc(_layer_body, hshape, pool1, W1.T, b1, ndeg)
    pool2 = seg32(h2, csF3, cdF, cntF)
    h3 = _tc(_layer_body, hshape, pool2, W2.T, b2, hdeg)
    pool3 = seg32(h3, csN3, cdN, cntN)
    h4 = _tc(_layer_body, hshape, pool3, W3.T, b3, ndeg)
    pool4 = seg32(h4, csF3, cdF, cntF)
    h5 = _tc(_layer_body, hshape, pool4, W4.T, b4, hdeg)
    pool5 = seg32(h5, csN3, cdN, cntN)
    h6 = _tc(_layer_body, hshape, pool5, W5.T, b5, ndeg)
    pool6 = seg32(h6, csF3, cdF, cntF)
    h7 = _tc(_layer_body, jax.ShapeDtypeStruct((_N, 1), _f32),
             pool6, W6.T, b6, hdeg)
    pool7 = seg1(h7.reshape(_N), csN3.reshape(_NW, _MAXE), cdN, cntN)

    c1m = conv1_w[:, 0, :].T
    h8, y = _tc(_head_a_body,
                (jax.ShapeDtypeStruct((_N, 1), _f32),
                 jax.ShapeDtypeStruct((_N, 16), _f32)),
                pool7.reshape(_N, 1), W7.T, b7, ndeg, h2, h4, h6,
                c1m[0:32], c1m[32:64], c1m[64:96], c1m[96:97], conv1_b)

    scm = h8.reshape(_G, _NPG)
    y3 = y.reshape(_G, _NPG, 16)
    w2r = jnp.transpose(conv2_w, (2, 1, 0)).reshape(80, 32)
    wd = jnp.transpose(out_w.reshape(2, 32, 11), (2, 1, 0)).reshape(352, 2)
    return _tc(_head_b_body, jax.ShapeDtypeStruct((_G, 2), _f32),
               scm, y3, w2r, conv2_b, wd, out_b,
               scratch_shapes=[pltpu.VMEM((15 * _G, 16), _f32)])


# speedup vs baseline: 2.1108x; 1.0235x over previous
import functools

import jax
import jax.numpy as jnp
from jax import lax
from jax.experimental import pallas as pl
from jax.experimental.pallas import tpu as pltpu
from jax.experimental.pallas import tpu_sc as plsc

_N = 10000
_E = 320000
_G = 100
_NPG = 100
_K = 30

_NC, _NS = 2, 16
_NW = _NC * _NS
_RNG = 312
_ACCR = 336
_DUM = 328
_MAXE = 12288
_NCHK = _MAXE // 128
_CHE = 2000
_NSC = _E // _CHE

_f32 = jnp.float32
_i32 = jnp.int32

_SC_PARAMS = pltpu.CompilerParams(use_tc_tiling_on_sc=False,
                                  needs_layout_passes=False)


def _mesh():
    return plsc.VectorSubcoreMesh(core_axis_name="c", subcore_axis_name="s",
                                  num_cores=_NC, num_subcores=_NS)


def _worker_bounds():
    c = lax.axis_index("c")
    s = lax.axis_index("s")
    w = c * _NS + s
    lo = w * _RNG
    hi = jnp.where(w == _NW - 1, _N, lo + _RNG)
    return w, lo, hi


def _compact_body(srch, dsth, csF, cdF, cntF, csN, cdN, cntN, hdeg, ndeg,
                  sbuf, dbuf, csF_v, cdF_v, csN_v, cdN_v, degF, degN, cnt_v):
    w, lo, hi = _worker_bounds()
    ii = lax.iota(_i32, 16)
    z16 = jnp.zeros((16,), _f32)
    ones16 = z16 + 1.0

    def zero_deg(i, _):
        degF[pl.ds(i * 16, 16)] = z16
        degN[pl.ds(i * 16, 16)] = z16
        return 0

    lax.fori_loop(0, _ACCR // 16, zero_deg, 0)

    def scan_chunk(kc, ptrs):
        pltpu.sync_copy(srch.at[pl.ds(kc * _CHE, _CHE)], sbuf)
        pltpu.sync_copy(dsth.at[pl.ds(kc * _CHE, _CHE)], dbuf)

        def scan_vreg(j, ptrs):
            pF, pN = ptrs
            n16 = sbuf[pl.ds(j * 16, 16)]
            h16 = dbuf[pl.ds(j * 16, 16)]
            mF = (h16 >= lo) & (h16 < hi)
            hloc = h16 - lo
            pFu = jnp.minimum(pF, _MAXE - 16)
            plsc.store_compressed(csF_v.at[pl.ds(pFu, 16)], n16, mask=mF)
            plsc.store_compressed(cdF_v.at[pl.ds(pFu, 16)], hloc, mask=mF)
            plsc.addupdate_scatter(degF, [hloc], ones16, mask=mF)
            pF = pF + lax.reduce_max(plsc.all_reduce_population_count(mF),
                                     axes=(0,))
            mN = (n16 >= lo) & (n16 < hi)
            nloc = n16 - lo
            pNu = jnp.minimum(pN, _MAXE - 16)
            plsc.store_compressed(csN_v.at[pl.ds(pNu, 16)], h16, mask=mN)
            plsc.store_compressed(cdN_v.at[pl.ds(pNu, 16)], nloc, mask=mN)
            plsc.addupdate_scatter(degN, [nloc], ones16, mask=mN)
            pN = pN + lax.reduce_max(plsc.all_reduce_population_count(mN),
                                     axes=(0,))
            return (pF, pN)

        return lax.fori_loop(0, _CHE // 16, scan_vreg, ptrs)

    pF, pN = lax.fori_loop(0, _NSC, scan_chunk, (jnp.int32(0), jnp.int32(0)))

    zi16 = ii * 0
    dum16 = zi16 + _DUM
    for (p, cs_v, cd_v, cnt_h) in ((pF, csF_v, cdF_v, cntF),
                                   (pN, csN_v, cdN_v, cntN)):
        pe = jnp.minimum(p, _MAXE - 256)
        for t in range(16):
            cs_v[pl.ds(pe + t * 16, 16)] = zi16
            cd_v[pl.ds(pe + t * 16, 16)] = dum16
        nch = ((pe + 256) // 256) * 2
        cnt_v[...] = jnp.broadcast_to(nch, (16,))
        pltpu.sync_copy(cnt_v, cnt_h.at[w])

    pltpu.sync_copy(csF_v, csF.at[w])
    pltpu.sync_copy(cdF_v, cdF.at[w])
    pltpu.sync_copy(csN_v, csN.at[w])
    pltpu.sync_copy(cdN_v, cdN.at[w])
    pltpu.sync_copy(degF.at[pl.ds(0, _RNG)], hdeg.at[pl.ds(lo, _RNG)])
    pltpu.sync_copy(degN.at[pl.ds(0, _RNG)], ndeg.at[pl.ds(lo, _RNG)])

    @pl.when(w == _NW - 1)
    def _():
        t0 = _RNG * _NW
        pltpu.sync_copy(degF.at[pl.ds(_RNG, 16)], hdeg.at[pl.ds(t0, 16)])
        pltpu.sync_copy(degN.at[pl.ds(_RNG, 16)], ndeg.at[pl.ds(t0, 16)])


@functools.lru_cache(maxsize=None)
def _make_compact():
    out_type = [
        jax.ShapeDtypeStruct((_NW, _MAXE), _i32),
        jax.ShapeDtypeStruct((_NW, _MAXE), _i32),
        jax.ShapeDtypeStruct((_NW, 16), _i32),
        jax.ShapeDtypeStruct((_NW, _MAXE), _i32),
        jax.ShapeDtypeStruct((_NW, _MAXE), _i32),
        jax.ShapeDtypeStruct((_NW, 16), _i32),
        jax.ShapeDtypeStruct((_N,), _f32),
        jax.ShapeDtypeStruct((_N,), _f32),
    ]
    scratch = [
        pltpu.VMEM((_CHE,), _i32),
        pltpu.VMEM((_CHE,), _i32),
        pltpu.VMEM((_MAXE,), _i32),
        pltpu.VMEM((_MAXE,), _i32),
        pltpu.VMEM((_MAXE,), _i32),
        pltpu.VMEM((_MAXE,), _i32),
        pltpu.VMEM((_ACCR,), _f32),
        pltpu.VMEM((_ACCR,), _f32),
        pltpu.VMEM((16,), _i32),
    ]
    return functools.partial(pl.kernel, out_type=out_type, mesh=_mesh(),
                             scratch_types=scratch,
                             compiler_params=_SC_PARAMS)(_compact_body)


def _seg_body(width, table, csrc3, cdst, cnth, out,
               csrc_v, cdst_v, cnt_v, rows0, rows1, acc, sem0, sem1):
    w, lo, hi = _worker_bounds()
    ii = lax.iota(_i32, 16)
    z16 = jnp.zeros((16,), _f32)

    pltpu.sync_copy(csrc3.at[w], csrc_v)
    pltpu.sync_copy(cdst.at[w], cdst_v)
    pltpu.sync_copy(cnth.at[w], cnt_v)
    nch = lax.reduce_max(cnt_v[...], axes=(0,))
    npair = nch // 2

    def zero_acc(r, _):
        for t in range(width // 16):
            acc[r, pl.ds(t * 16, 16)] = z16
        return 0

    lax.fori_loop(0, _ACCR, zero_acc, 0)

    def process(kc, rows_v):
        for v in range(8):
            dloc = cdst_v[pl.ds(kc * 128 + v * 16, 16)]
            rbase = ii + v * 16
            if width == 32:
                for j in range(width):
                    j16 = ii * 0 + j
                    vals = plsc.load_gather(rows_v, [rbase, j16])
                    plsc.addupdate_scatter(acc, [dloc, j16], vals)
            else:

                def cols(jg, _):
                    for t in range(16):
                        j16 = ii * 0 + (jg * 16 + t)
                        vals = plsc.load_gather(rows_v, [rbase, j16])
                        plsc.addupdate_scatter(acc, [dloc, j16], vals)
                    return 0

                lax.fori_loop(0, width // 16, cols, 0)

    pltpu.async_copy(table.at[csrc_v.at[0]], rows0, sem0)

    def pair(kp, _):
        kc = kp * 2
        pltpu.make_async_copy(table.at[csrc_v.at[kc]], rows0, sem0).wait()
        pltpu.async_copy(table.at[csrc_v.at[kc + 1]], rows1, sem1)
        process(kc, rows0)
        pltpu.make_async_copy(table.at[csrc_v.at[kc + 1]], rows1, sem1).wait()

        @pl.when(kp + 1 < npair)
        def _():
            pltpu.async_copy(table.at[csrc_v.at[kc + 2]], rows0, sem0)

        process(kc + 1, rows1)
        return 0

    lax.fori_loop(0, npair, pair, 0)

    pltpu.sync_copy(acc.at[pl.ds(0, _RNG)], out.at[pl.ds(lo, _RNG)])

    @pl.when(w == _NW - 1)
    def _():
        t0 = _RNG * _NW
        pltpu.sync_copy(acc.at[pl.ds(_RNG, 16)], out.at[pl.ds(t0, 16)])


def _seg1_body(table, csrcf, cdst, cnth, out,
               tbl_v, csrc_v, cdst_v, cnt_v, acc):
    w, lo, hi = _worker_bounds()
    ii = lax.iota(_i32, 16)
    z16 = jnp.zeros((16,), _f32)

    pltpu.sync_copy(table, tbl_v)
    pltpu.sync_copy(csrcf.at[w], csrc_v)
    pltpu.sync_copy(cdst.at[w], cdst_v)
    pltpu.sync_copy(cnth.at[w], cnt_v)
    nch = lax.reduce_max(cnt_v[...], axes=(0,))

    def zero_acc(r, _):
        acc[pl.ds(r * 16, 16)] = z16
        return 0

    lax.fori_loop(0, _ACCR // 16, zero_acc, 0)

    def chunk(kc, _):
        for v in range(8):
            s16 = csrc_v[pl.ds(kc * 128 + v * 16, 16)]
            dloc = cdst_v[pl.ds(kc * 128 + v * 16, 16)]
            vals = plsc.load_gather(tbl_v, [s16])
            plsc.addupdate_scatter(acc, [dloc], vals)
        return 0

    lax.fori_loop(0, nch, chunk, 0)

    pltpu.sync_copy(acc.at[pl.ds(0, _RNG)], out.at[pl.ds(lo, _RNG)])

    @pl.when(w == _NW - 1)
    def _():
        t0 = _RNG * _NW
        pltpu.sync_copy(acc.at[pl.ds(_RNG, 16)], out.at[pl.ds(t0, 16)])


@functools.lru_cache(maxsize=None)
def _make_seg(width):
    if width == 1:
        scratch = [
            pltpu.VMEM((_N,), _f32),
            pltpu.VMEM((_MAXE,), _i32),
            pltpu.VMEM((_MAXE,), _i32),
            pltpu.VMEM((16,), _i32),
            pltpu.VMEM((_ACCR,), _f32),
        ]
        return functools.partial(
            pl.kernel, out_type=jax.ShapeDtypeStruct((_N,), _f32),
            mesh=_mesh(), scratch_types=scratch,
            compiler_params=_SC_PARAMS)(_seg1_body)
    scratch = [
        pltpu.VMEM((_NCHK, 128), _i32),
        pltpu.VMEM((_MAXE,), _i32),
        pltpu.VMEM((16,), _i32),
        pltpu.VMEM((128, width), _f32),
        pltpu.VMEM((128, width), _f32),
        pltpu.VMEM((_ACCR, width), _f32),
        pltpu.SemaphoreType.DMA,
        pltpu.SemaphoreType.DMA,
    ]
    return functools.partial(
        pl.kernel, out_type=jax.ShapeDtypeStruct((_N, width), _f32),
        mesh=_mesh(), scratch_types=scratch,
        compiler_params=_SC_PARAMS)(functools.partial(_seg_body, width))


def _tc(fn, out_shape, *args, scratch_shapes=()):
    return pl.pallas_call(fn, out_shape=out_shape,
                          scratch_shapes=list(scratch_shapes))(*args)


def _layer_body(pool, wt, b, deg, h_out):
    sizes = deg[...][:, None] + 1.0
    p = jnp.dot(pool[...], wt[...], preferred_element_type=_f32)
    h_out[...] = jnp.tanh((p + b[...][None, :]) / sizes)


def _head_a_body(pool7, w7t, b7, ndeg, h2, h4, h6, c1a, c1b_, c1c, c1d, cb,
                 h8_out, y_out):
    ns = ndeg[...][:, None] + 1.0
    p = pool7[...] * w7t[...]
    h8 = jnp.tanh((p + b7[...][None, :]) / ns)
    h8_out[...] = h8
    y = (jnp.dot(h2[...], c1a[...], preferred_element_type=_f32)
         + jnp.dot(h4[...], c1b_[...], preferred_element_type=_f32)
         + jnp.dot(h6[...], c1c[...], preferred_element_type=_f32)
         + h8 * c1d[...]
         + cb[...][None, :])
    y_out[...] = jnp.maximum(y, 0.0)


def _head_b_body(scm, y3, w2r, b2, wd, ob, out, p1s):
    vals = scm[...]
    y = y3[...]
    ii = lax.broadcasted_iota(_i32, (_G, _NPG), 1)
    tmp = None
    for k in range(_K):
        mx = jnp.max(vals, axis=1, keepdims=True)
        sel = jnp.min(jnp.where(vals == mx, ii, _NPG), axis=1, keepdims=True)
        m = (ii == sel)
        ysel = jnp.sum(m.astype(_f32)[:, :, None] * y, axis=1)
        vals = jnp.where(m, -2.0, vals)
        if k % 2 == 0:
            tmp = ysel
        else:
            j = k // 2
            p1s[(j * _G):(j + 1) * _G, :] = jnp.maximum(tmp, ysel)
    c2 = jnp.zeros((11 * _G, 32), _f32)
    for t in range(5):
        st = p1s[(t * _G):(t * _G + 11 * _G), :]
        c2 = c2 + jnp.dot(st, w2r[(t * 16):(t + 1) * 16, :],
                          preferred_element_type=_f32)
    c2 = jnp.maximum(c2 + b2[...][None, :], 0.0)
    acc = jnp.zeros((_G, 2), _f32)
    for mm in range(11):
        acc = acc + jnp.dot(c2[(mm * _G):(mm + 1) * _G, :],
                            wd[(mm * 32):(mm + 1) * 32, :],
                            preferred_element_type=_f32)
    out[...] = jnp.maximum(acc + ob[...][None, :], 0.0)


def kernel(node_feat, W0, b0, W1, b1, W2, b2, W3, b3, W4, b4, W5, b5,
           W6, b6, W7, b7, conv1_w, conv1_b, conv2_w, conv2_b,
           out_w, out_b, inc_node, inc_hedge):
    src = inc_node.astype(_i32)
    dst = inc_hedge.astype(_i32)

    (csF, cdF, cntF, csN, cdN, cntN, hdeg, ndeg) = _make_compact()(src, dst)
    csF3 = csF.reshape(_NW, _NCHK, 128)
    csN3 = csN.reshape(_NW, _NCHK, 128)

    seg128 = _make_seg(128)
    seg32 = _make_seg(32)
    seg1 = _make_seg(1)

    hshape = jax.ShapeDtypeStruct((_N, 32), _f32)

    pool0 = seg128(node_feat, csF3, cdF, cntF)
    h1 = _tc(_layer_body, hshape, pool0, W0.T, b0, hdeg)
    pool1 = seg32(h1, csN3, cdN, cntN)
    h2 = _tc(_layer_body, hshape, pool1, W1.T, b1, ndeg)
    pool2 = seg32(h2, csF3, cdF, cntF)
    h3 = _tc(_layer_body, hshape, pool2, W2.T, b2, hdeg)
    pool3 = seg32(h3, csN3, cdN, cntN)
    h4 = _tc(_layer_body, hshape, pool3, W3.T, b3, ndeg)
    pool4 = seg32(h4, csF3, cdF, cntF)
    h5 = _tc(_layer_body, hshape, pool4, W4.T, b4, hdeg)
    pool5 = seg32(h5, csN3, cdN, cntN)
    h6 = _tc(_layer_body, hshape, pool5, W5.T, b5, ndeg)
    pool6 = seg32(h6, csF3, cdF, cntF)
    h7 = _tc(_layer_body, jax.ShapeDtypeStruct((_N, 1), _f32),
             pool6, W6.T, b6, hdeg)
    pool7 = seg1(h7.reshape(_N), csN3.reshape(_NW, _MAXE), cdN, cntN)

    c1m = conv1_w[:, 0, :].T
    h8, y = _tc(_head_a_body,
                (jax.ShapeDtypeStruct((_N, 1), _f32),
                 jax.ShapeDtypeStruct((_N, 16), _f32)),
                pool7.reshape(_N, 1), W7.T, b7, ndeg, h2, h4, h6,
                c1m[0:32], c1m[32:64], c1m[64:96], c1m[96:97], conv1_b)

    scm = h8.reshape(_G, _NPG)
    y3 = y.reshape(_G, _NPG, 16)
    w2r = jnp.transpose(conv2_w, (2, 1, 0)).reshape(80, 32)
    wd = jnp.transpose(out_w.reshape(2, 32, 11), (2, 1, 0)).reshape(352, 2)
    return _tc(_head_b_body, jax.ShapeDtypeStruct((_G, 2), _f32),
               scm, y3, w2r, conv2_b, wd, out_b,
               scratch_shapes=[pltpu.VMEM((15 * _G, 16), _f32)])
